# speedup vs baseline: 3.8411x; 3.8411x over previous
import functools
import math

import jax
import jax.numpy as jnp
from jax import lax
from jax.experimental import pallas as pl
from jax.experimental.pallas import tpu as pltpu
from jax.experimental.pallas import tpu_sc as plsc

D_MODEL = 128
SEQ = 200
BATCH = 4096
SCALE = math.sqrt(float(D_MODEL))

_info = plsc.get_sparse_core_info()
NC = _info.num_cores
NS = _info.num_subcores
LANES = _info.num_lanes
NW = NC * NS
SEQ_PER_W = BATCH // NW


def _sc_kernel(tok_hbm, table_hbm, pos_hbm, out_hbm, idx_v, rows_v, pos_v, sem):
    wid = lax.axis_index("s") * NC + lax.axis_index("c")
    pltpu.sync_copy(pos_hbm.at[pl.ds(0, SEQ)], pos_v)

    def seq_body(i, carry):
        base = (wid * SEQ_PER_W + i) * SEQ
        pltpu.sync_copy(tok_hbm.at[pl.ds(base, SEQ)], idx_v)
        pltpu.async_copy(table_hbm.at[idx_v.at[pl.ds(0, 104)]],
                         rows_v.at[pl.ds(0, 104)], sem).wait()
        pltpu.async_copy(table_hbm.at[idx_v.at[pl.ds(104, 96)]],
                         rows_v.at[pl.ds(104, 96)], sem).wait()

        def row_body(t, c):
            for j in range(D_MODEL // LANES):
                sl = pl.ds(j * LANES, LANES)
                rows_v[t, sl] = rows_v[t, sl] * SCALE + pos_v[t, sl]
            return c

        lax.fori_loop(0, SEQ, row_body, 0)
        pltpu.sync_copy(rows_v, out_hbm.at[pl.ds(base, SEQ)])
        return carry

    lax.fori_loop(0, SEQ_PER_W, seq_body, 0)


def kernel(tokens, table, pos):
    batch, seq = tokens.shape
    flat_tok = tokens.reshape(batch * seq)
    mesh = plsc.VectorSubcoreMesh(core_axis_name="c", subcore_axis_name="s")
    run = functools.partial(
        pl.kernel,
        mesh=mesh,
        out_type=jax.ShapeDtypeStruct((batch * seq, D_MODEL), jnp.float32),
        scratch_types=[
            pltpu.VMEM((SEQ,), jnp.int32),
            pltpu.VMEM((SEQ, D_MODEL), jnp.float32),
            pltpu.VMEM((SEQ, D_MODEL), jnp.float32),
            pltpu.SemaphoreType.DMA,
        ],
    )(_sc_kernel)
    out = run(flat_tok, table, pos)
    return out.reshape(batch, seq, D_MODEL)


# speedup vs baseline: 7.4777x; 1.9468x over previous
import functools
import math

import jax
import jax.numpy as jnp
from jax import lax
from jax.experimental import pallas as pl
from jax.experimental.pallas import tpu as pltpu
from jax.experimental.pallas import tpu_sc as plsc

D_MODEL = 128
SEQ = 200
BATCH = 4096
SCALE = math.sqrt(float(D_MODEL))

_info = plsc.get_sparse_core_info()
NC = _info.num_cores
NS = _info.num_subcores
LANES = _info.num_lanes
NW = NC * NS
SEQ_PER_W = BATCH // NW
NBUF = 3
G0, G1 = 104, SEQ - 104


def _sc_kernel(tok_hbm, table_hbm, pos_hbm, out_hbm,
               idx_all, pos_v, rows0, rows1, rows2,
               sg0, sg1, sg2, so0, so1, so2):
    rows = (rows0, rows1, rows2)
    sg = (sg0, sg1, sg2)
    so = (so0, so1, so2)
    wid = lax.axis_index("s") * NC + lax.axis_index("c")
    tok_base = wid * (SEQ_PER_W * SEQ)

    pltpu.sync_copy(tok_hbm.at[pl.ds(tok_base, SEQ_PER_W * SEQ)], idx_all)
    pltpu.sync_copy(pos_hbm.at[pl.ds(0, SEQ)], pos_v)

    def gather_start(c, b):
        off = c * SEQ
        pltpu.make_async_copy(table_hbm.at[idx_all.at[pl.ds(off, G0)]],
                              rows[b].at[pl.ds(0, G0)], sg[b]).start()
        pltpu.make_async_copy(table_hbm.at[idx_all.at[pl.ds(off + G0, G1)]],
                              rows[b].at[pl.ds(G0, G1)], sg[b]).start()

    def gather_wait(c, b):
        off = c * SEQ
        pltpu.make_async_copy(table_hbm.at[idx_all.at[pl.ds(off, G0)]],
                              rows[b].at[pl.ds(0, G0)], sg[b]).wait()
        pltpu.make_async_copy(table_hbm.at[idx_all.at[pl.ds(off + G0, G1)]],
                              rows[b].at[pl.ds(G0, G1)], sg[b]).wait()

    def out_start(c, b):
        pltpu.make_async_copy(rows[b], out_hbm.at[pl.ds(tok_base + c * SEQ, SEQ)],
                              so[b]).start()

    def out_wait(c, b):
        pltpu.make_async_copy(rows[b], out_hbm.at[pl.ds(tok_base + c * SEQ, SEQ)],
                              so[b]).wait()

    def compute(b):
        buf = rows[b]

        def row_body(t, carry):
            for tt in range(2):
                for j in range(D_MODEL // LANES):
                    sl = pl.ds(j * LANES, LANES)
                    buf[t * 2 + tt, sl] = buf[t * 2 + tt, sl] * SCALE + pos_v[t * 2 + tt, sl]
            return carry

        lax.fori_loop(0, SEQ // 2, row_body, 0)

    gather_start(0, 0)
    gather_start(1, 1)

    MAIN = SEQ_PER_W - 2

    def pipe_body(g, carry):
        for b in range(NBUF):
            c = g * NBUF + b
            bn = (b + 2) % NBUF

            @pl.when(jnp.logical_or(g > 0, b > 0))
            def _():
                out_wait(c - 1, bn)

            gather_start(c + 2, bn)
            gather_wait(c, b)
            compute(b)
            out_start(c, b)
        return carry

    lax.fori_loop(0, MAIN // NBUF, pipe_body, 0)

    for c in (MAIN, MAIN + 1):
        b = c % NBUF
        out_wait(c - 1, (b + 2) % NBUF)
        gather_wait(c, b)
        compute(b)
        out_start(c, b)
    out_wait(MAIN + 1, (MAIN + 1) % NBUF)


def kernel(tokens, table, pos):
    batch, seq = tokens.shape
    flat_tok = tokens.reshape(batch * seq)
    mesh = plsc.VectorSubcoreMesh(core_axis_name="c", subcore_axis_name="s")
    run = functools.partial(
        pl.kernel,
        mesh=mesh,
        out_type=jax.ShapeDtypeStruct((batch * seq, D_MODEL), jnp.float32),
        scratch_types=[
            pltpu.VMEM((SEQ_PER_W * SEQ,), jnp.int32),
            pltpu.VMEM((SEQ, D_MODEL), jnp.float32),
            pltpu.VMEM((SEQ, D_MODEL), jnp.float32),
            pltpu.VMEM((SEQ, D_MODEL), jnp.float32),
            pltpu.VMEM((SEQ, D_MODEL), jnp.float32),
            pltpu.SemaphoreType.DMA,
            pltpu.SemaphoreType.DMA,
            pltpu.SemaphoreType.DMA,
            pltpu.SemaphoreType.DMA,
            pltpu.SemaphoreType.DMA,
            pltpu.SemaphoreType.DMA,
        ],
    )(_sc_kernel)
    out = run(flat_tok, table, pos)
    return out.reshape(batch, seq, D_MODEL)


# speedup vs baseline: 9.0363x; 1.2084x over previous
import functools
import math

import jax
import jax.numpy as jnp
from jax import lax
from jax.experimental import pallas as pl
from jax.experimental.pallas import tpu as pltpu
from jax.experimental.pallas import tpu_sc as plsc

D_MODEL = 128
SEQ = 200
BATCH = 4096
SCALE = math.sqrt(float(D_MODEL))

_info = plsc.get_sparse_core_info()
NC = _info.num_cores
NS = _info.num_subcores
LANES = _info.num_lanes
NW = NC * NS
SEQ_PER_W = BATCH // NW
NBUF = 3
G0, G1 = 104, SEQ - 104


def _sc_kernel(tok_hbm, table_hbm, pos_hbm, out_hbm,
               idx_all, pos_v, rows0, rows1, rows2,
               sg0, sg1, sg2, so0, so1, so2):
    rows = (rows0, rows1, rows2)
    sg = (sg0, sg1, sg2)
    so = (so0, so1, so2)
    wid = lax.axis_index("s") * NC + lax.axis_index("c")
    tok_base = wid * (SEQ_PER_W * SEQ)

    pltpu.sync_copy(tok_hbm.at[pl.ds(tok_base, SEQ_PER_W * SEQ)], idx_all)
    pltpu.sync_copy(pos_hbm.at[pl.ds(0, SEQ)], pos_v)

    def gather_start(c, b):
        off = c * SEQ
        pltpu.make_async_copy(table_hbm.at[idx_all.at[pl.ds(off, G0)]],
                              rows[b].at[pl.ds(0, G0)], sg[b]).start()
        pltpu.make_async_copy(table_hbm.at[idx_all.at[pl.ds(off + G0, G1)]],
                              rows[b].at[pl.ds(G0, G1)], sg[b]).start()

    def gather_wait(c, b):
        off = c * SEQ
        pltpu.make_async_copy(table_hbm.at[idx_all.at[pl.ds(off, G0)]],
                              rows[b].at[pl.ds(0, G0)], sg[b]).wait()
        pltpu.make_async_copy(table_hbm.at[idx_all.at[pl.ds(off + G0, G1)]],
                              rows[b].at[pl.ds(G0, G1)], sg[b]).wait()

    def out_start(c, b):
        pltpu.make_async_copy(rows[b], out_hbm.at[pl.ds(tok_base + c * SEQ, SEQ)],
                              so[b]).start()

    def out_wait(c, b):
        pltpu.make_async_copy(rows[b], out_hbm.at[pl.ds(tok_base + c * SEQ, SEQ)],
                              so[b]).wait()

    def compute(b):
        buf = rows[b]

        def row_body(t, carry):
            for tt in range(4):
                for j in range(D_MODEL // LANES):
                    sl = pl.ds(j * LANES, LANES)
                    buf[t * 4 + tt, sl] = buf[t * 4 + tt, sl] * SCALE + pos_v[t * 4 + tt, sl]
            return carry

        lax.fori_loop(0, SEQ // 4, row_body, 0)

    gather_start(0, 0)
    gather_start(1, 1)

    MAIN = SEQ_PER_W - 2

    def pipe_body(g, carry):
        for b in range(NBUF):
            c = g * NBUF + b
            bn = (b + 2) % NBUF

            gather_wait(c, b)
            compute(b)
            out_start(c, b)

            @pl.when(jnp.logical_or(g > 0, b > 0))
            def _():
                out_wait(c - 1, bn)

            gather_start(c + 2, bn)
        return carry

    lax.fori_loop(0, MAIN // NBUF, pipe_body, 0)

    for c in (MAIN, MAIN + 1):
        b = c % NBUF
        gather_wait(c, b)
        compute(b)
        out_start(c, b)
        out_wait(c - 1, (b + 2) % NBUF)
    out_wait(MAIN + 1, (MAIN + 1) % NBUF)


def kernel(tokens, table, pos):
    batch, seq = tokens.shape
    flat_tok = tokens.reshape(batch * seq)
    mesh = plsc.VectorSubcoreMesh(core_axis_name="c", subcore_axis_name="s")
    run = functools.partial(
        pl.kernel,
        mesh=mesh,
        out_type=jax.ShapeDtypeStruct((batch * seq, D_MODEL), jnp.float32),
        scratch_types=[
            pltpu.VMEM((SEQ_PER_W * SEQ,), jnp.int32),
            pltpu.VMEM((SEQ, D_MODEL), jnp.float32),
            pltpu.VMEM((SEQ, D_MODEL), jnp.float32),
            pltpu.VMEM((SEQ, D_MODEL), jnp.float32),
            pltpu.VMEM((SEQ, D_MODEL), jnp.float32),
            pltpu.SemaphoreType.DMA,
            pltpu.SemaphoreType.DMA,
            pltpu.SemaphoreType.DMA,
            pltpu.SemaphoreType.DMA,
            pltpu.SemaphoreType.DMA,
            pltpu.SemaphoreType.DMA,
        ],
    )(_sc_kernel)
    out = run(flat_tok, table, pos)
    return out.reshape(batch, seq, D_MODEL)
